# Initial kernel scaffold; baseline (speedup 1.0000x reference)
import jax
import jax.numpy as jnp
from jax.experimental import pallas as pl


def kernel(box_cls, box_regression, anchors):
    raise NotImplementedError("write your pallas kernel here")


# speedup vs baseline: 4.7871x; 4.7871x over previous
import jax
import jax.numpy as jnp
import numpy as np
from jax.experimental import pallas as pl
from jax.experimental.pallas import tpu as pltpu

_N, _A, _W, _H, _D = 2, 3, 32, 32, 32
_NB = 6
_RES = 160.0
_PRE = 2048
_POST = 512
_TH = 0.7
_MIN = 1e-3
_XCLIP = float(np.log(1000.0 / 16.0))


def _rpn_nms_kernel(dt_ref, at_ref, s_ref, out_ref):
    dt = dt_ref[0]
    at = at_ref[0]
    sc = s_ref[0]

    a1 = at[0:3, :]
    a2 = at[3:6, :]
    wdh = a2 - a1
    ctr = a1 + 0.5 * wdh
    pc = dt[0:3, :] * wdh + ctr
    psz = jnp.exp(jnp.minimum(dt[3:6, :], _XCLIP)) * wdh
    lo = jnp.clip(pc - 0.5 * psz, 0.0, _RES)
    hi = jnp.clip(pc + 0.5 * psz, 0.0, _RES)
    szs = hi - lo
    probs = jax.nn.sigmoid(sc)
    valid = ((szs[0:1] >= _MIN) & (szs[1:2] >= _MIN)
             & (szs[2:3] >= _MIN) & (probs > 0.0))
    vol = szs[0:1] * szs[1:2] * szs[2:3]

    x1, y1, z1 = lo[0:1], lo[1:2], lo[2:3]
    x2, y2, z2 = hi[0:1], hi[1:2], hi[2:3]
    lane = jax.lax.broadcasted_iota(jnp.int32, (1, _PRE), 1)
    l8 = jax.lax.broadcasted_iota(jnp.int32, (1, 8), 1)

    out_ref[...] = jnp.zeros((1, _POST, 8), jnp.float32)

    def _pick(v, i):
        return jnp.sum(jnp.where(lane == i, v, 0.0))

    def body(i, carry):
        supp, c = carry
        bx1 = _pick(x1, i); by1 = _pick(y1, i); bz1 = _pick(z1, i)
        bx2 = _pick(x2, i); by2 = _pick(y2, i); bz2 = _pick(z2, i)
        bv = _pick(vol, i); bp = _pick(probs, i)
        alive = _pick(supp, i) < 0.5
        ix = jnp.maximum(jnp.minimum(x2, bx2) - jnp.maximum(x1, bx1), 0.0)
        iy = jnp.maximum(jnp.minimum(y2, by2) - jnp.maximum(y1, by1), 0.0)
        iz = jnp.maximum(jnp.minimum(z2, bz2) - jnp.maximum(z1, bz1), 0.0)
        inter = ix * iy * iz
        iou = inter / (vol + bv - inter + 1e-9)
        mask = ((iou > _TH) & (lane > i)).astype(jnp.float32)
        supp = jnp.where(alive, jnp.maximum(supp, mask), supp)
        emit = alive & (c < _POST)

        @pl.when(emit)
        def _():
            row = jnp.where(l8 == 0, bx1,
                  jnp.where(l8 == 1, by1,
                  jnp.where(l8 == 2, bz1,
                  jnp.where(l8 == 3, bx2,
                  jnp.where(l8 == 4, by2,
                  jnp.where(l8 == 5, bz2,
                  jnp.where(l8 == 6, bp, 0.0)))))))
            out_ref[0, pl.ds(c, 1), :] = row

        return supp, c + jnp.where(emit, 1, 0).astype(jnp.int32)

    supp0 = 1.0 - valid.astype(jnp.float32)
    jax.lax.fori_loop(0, _PRE, body, (supp0, jnp.int32(0)))


def kernel(box_cls, box_regression, anchors):
    n = _N
    scores = (box_cls.reshape(n, _A, 1, _W, _H, _D)
              .transpose(0, 3, 4, 5, 1, 2).reshape(n, -1))
    deltas = (box_regression.reshape(n, _A, _NB, _W, _H, _D)
              .transpose(0, 3, 4, 5, 1, 2).reshape(n, -1, _NB))
    top_s, top_i = jax.lax.top_k(scores, _PRE)
    d = jnp.take_along_axis(deltas, top_i[..., None], axis=1)
    anc = jnp.take(anchors, top_i, axis=0)
    dt = d.transpose(0, 2, 1)
    at = anc.transpose(0, 2, 1)
    sc = top_s[:, None, :]

    out = pl.pallas_call(
        _rpn_nms_kernel,
        grid=(n,),
        in_specs=[
            pl.BlockSpec((1, 6, _PRE), lambda i: (i, 0, 0)),
            pl.BlockSpec((1, 6, _PRE), lambda i: (i, 0, 0)),
            pl.BlockSpec((1, 1, _PRE), lambda i: (i, 0, 0)),
        ],
        out_specs=pl.BlockSpec((1, _POST, 8), lambda i: (i, 0, 0)),
        out_shape=jax.ShapeDtypeStruct((n, _POST, 8), jnp.float32),
    )(dt, at, sc)
    return out[..., :7]
